# baseline (speedup 1.0000x reference)
import functools

import jax
import jax.numpy as jnp
from jax import lax
from jax.experimental import pallas as pl
from jax.experimental.pallas import tpu as pltpu
from jax.experimental.pallas import tpu_sc as plsc

B = 16384
D = 32
NC = 2
NS = 16
NW = NC * NS
L = 16
NR = 1000000
GB = 1024
NCHK = 31
HCAP = 768
CAP = 64
SW = 128


def _scan_body(user_hbm, item_hbm, utT_hbm, itT_hbm, ustage, istage,
               fu, fi, hpu, hru, hpi, hri, ubuf, ibuf, stag, cpos, crow,
               sem):
    wid = lax.axis_index("s") * NC + lax.axis_index("c")
    lo_g = 244 * wid + jnp.minimum(wid, 5)
    ng = jnp.where(wid < 5, 245, 244)
    lo_row = lo_g * 128
    hi_row = jnp.minimum((lo_g + ng) * 128, NR)

    pltpu.sync_copy(user_hbm, fu)
    pltpu.sync_copy(item_hbm, fi)

    neg1 = jnp.full((L,), -1, jnp.int32)
    zero = jnp.zeros((L,), jnp.int32)
    for j in range(HCAP // L):
        hru[pl.ds(j * L, L)] = neg1
        hri[pl.ds(j * L, L)] = neg1
    for j in range(CAP // L):
        crow[pl.ds(j * L, L)] = zero

    def bucket(kk, cnts):
        cu, ci = cnts
        ks = kk * L + lax.iota(jnp.int32, L)
        vu = fu[pl.ds(kk * L, L)]
        mu = jnp.logical_and(vu >= lo_row, vu < hi_row)
        plsc.store_compressed(hpu.at[pl.ds(cu, L)], ks, mask=mu)
        plsc.store_compressed(hru.at[pl.ds(cu, L)], vu, mask=mu)
        cu = cu + plsc.all_reduce_population_count(mu)[0]
        vi = fi[pl.ds(kk * L, L)]
        mi = jnp.logical_and(vi >= lo_row, vi < hi_row)
        plsc.store_compressed(hpi.at[pl.ds(ci, L)], ks, mask=mi)
        plsc.store_compressed(hri.at[pl.ds(ci, L)], vi, mask=mi)
        ci = ci + plsc.all_reduce_population_count(mi)[0]
        return cu, ci

    lax.fori_loop(0, B // L, bucket, (jnp.int32(0), jnp.int32(0)))

    def process(tbl_hbm, buf, hp, hr, stage, c0, c1, startp, wsize):
        for c in range(4):
            pltpu.sync_copy(tbl_hbm.at[c, slice(None), pl.ds(startp, wsize)],
                            buf.at[c, slice(None), pl.ds(0, wsize)])
        sent = jnp.full((L,), -1, jnp.int32)
        for j in range(CAP // L):
            cpos[pl.ds(j * L, L)] = sent

        def rescan(hv, cc):
            rows = hr[pl.ds(hv * L, L)]
            m = jnp.logical_and(rows >= c0, rows < c1)
            plsc.store_compressed(cpos.at[pl.ds(cc, L)],
                                  hp[pl.ds(hv * L, L)], mask=m)
            plsc.store_compressed(crow.at[pl.ds(cc, L)],
                                  rows - startp, mask=m)
            return cc + plsc.all_reduce_population_count(m)[0]

        lax.fori_loop(0, HCAP // L, rescan, jnp.int32(0))

        for grp in range(CAP // L):
            rows = crow[pl.ds(grp * L, L)]
            slot = grp * L + lax.iota(jnp.int32, L)
            for d in range(D):
                cvec = jnp.full((L,), d >> 3, jnp.int32)
                evec = jnp.full((L,), d & 7, jnp.int32)
                vals = plsc.load_gather(buf, [cvec, evec, rows])
                plsc.store_scatter(stag, [slot, jnp.full((L,), d, jnp.int32)],
                                   vals)
        pltpu.async_copy(
            stag, stage.at[plsc.Indices(cpos, ignored_value=-1)], sem).wait()

    def chunk(ch, carry):
        c0 = lo_row + ch * GB
        c1 = jnp.minimum(c0 + GB, jnp.minimum(hi_row, 999936))
        startp = pl.multiple_of(jnp.minimum(c0, 998912), 128)
        process(utT_hbm, ubuf, hpu, hru, ustage, c0, c1, startp, GB)
        process(itT_hbm, ibuf, hpi, hri, istage, c0, c1, startp, GB)
        return carry

    lax.fori_loop(0, NCHK, chunk, 0)

    t0 = jnp.minimum(jnp.int32(999936), hi_row)
    ts = pl.multiple_of(jnp.int32(999936), 128)
    process(utT_hbm, ubuf, hpu, hru, ustage, t0, hi_row, ts, 128)
    process(itT_hbm, ibuf, hpi, hri, istage, t0, hi_row, ts, 128)


@functools.partial(
    pl.kernel,
    out_type=[jax.ShapeDtypeStruct((B, SW), jnp.float32),
              jax.ShapeDtypeStruct((B, SW), jnp.float32)],
    mesh=plsc.VectorSubcoreMesh(core_axis_name="c", subcore_axis_name="s"),
    compiler_params=pltpu.CompilerParams(needs_layout_passes=False),
    scratch_types=[
        pltpu.VMEM((B,), jnp.int32),
        pltpu.VMEM((B,), jnp.int32),
        pltpu.VMEM((HCAP,), jnp.int32),
        pltpu.VMEM((HCAP,), jnp.int32),
        pltpu.VMEM((HCAP,), jnp.int32),
        pltpu.VMEM((HCAP,), jnp.int32),
        pltpu.VMEM((4, 8, GB), jnp.float32),
        pltpu.VMEM((4, 8, GB), jnp.float32),
        pltpu.VMEM((CAP, SW), jnp.float32),
        pltpu.VMEM((CAP,), jnp.int32),
        pltpu.VMEM((CAP,), jnp.int32),
        pltpu.SemaphoreType.DMA,
    ],
)
def _scan_kernel(user_hbm, item_hbm, utT_hbm, itT_hbm, ustage, istage,
                 fu, fi, hpu, hru, hpi, hri, ubuf, ibuf, stag, cpos, crow,
                 sem):
    _scan_body(user_hbm, item_hbm, utT_hbm, itT_hbm, ustage, istage,
               fu, fi, hpu, hru, hpi, hri, ubuf, ibuf, stag, cpos, crow,
               sem)


def _dot_body(u_ref, v_ref, o_ref):
    o_ref[...] = jnp.sum(u_ref[:, :D] * v_ref[:, :D], axis=1)


_dot_kernel = pl.pallas_call(
    _dot_body,
    out_shape=jax.ShapeDtypeStruct((B,), jnp.float32),
    grid=(16,),
    in_specs=[
        pl.BlockSpec((B // 16, SW), lambda i: (i, 0)),
        pl.BlockSpec((B // 16, SW), lambda i: (i, 0)),
    ],
    out_specs=pl.BlockSpec((B // 16,), lambda i: (i,)),
)


def kernel(user, item, user_table, item_table):
    u = user.astype(jnp.int32)
    it = item.astype(jnp.int32)
    utT = user_table.T.reshape(4, 8, NR)
    itT = item_table.T.reshape(4, 8, NR)
    ustage, istage = _scan_kernel(u, it, utT, itT)
    return _dot_kernel(ustage, istage)


# speedup vs baseline: 1.4747x; 1.4747x over previous
import functools

import jax
import jax.numpy as jnp
from jax import lax
from jax.experimental import pallas as pl
from jax.experimental.pallas import tpu as pltpu
from jax.experimental.pallas import tpu_sc as plsc

B = 16384
D = 32
NC = 2
NS = 16
NW = NC * NS
L = 16
NR = 1000000
GB = 1024
NCHK = 31
HCAP = 768
CAP = 64
SW = 128


def _scan_body(user_hbm, item_hbm, utT_hbm, itT_hbm, ustage, istage,
               fu, fi, hpu, hru, hpi, hri, ubuf, ibuf, stag, cpos, crow,
               sem):
    wid = lax.axis_index("s") * NC + lax.axis_index("c")
    lo_g = 244 * wid + jnp.minimum(wid, 5)
    ng = jnp.where(wid < 5, 245, 244)
    lo_row = lo_g * 128
    hi_row = jnp.minimum((lo_g + ng) * 128, NR)

    pltpu.sync_copy(user_hbm, fu)
    pltpu.sync_copy(item_hbm, fi)

    neg1 = jnp.full((L,), -1, jnp.int32)
    zero = jnp.zeros((L,), jnp.int32)
    for j in range(HCAP // L):
        hru[pl.ds(j * L, L)] = neg1
        hri[pl.ds(j * L, L)] = neg1
    for j in range(CAP // L):
        crow[pl.ds(j * L, L)] = zero

    def bucket(kk, cnts):
        cu, ci = cnts
        ks = kk * L + lax.iota(jnp.int32, L)
        vu = fu[pl.ds(kk * L, L)]
        mu = jnp.logical_and(vu >= lo_row, vu < hi_row)
        plsc.store_compressed(hpu.at[pl.ds(cu, L)], ks, mask=mu)
        plsc.store_compressed(hru.at[pl.ds(cu, L)], vu, mask=mu)
        cu = cu + plsc.all_reduce_population_count(mu)[0]
        vi = fi[pl.ds(kk * L, L)]
        mi = jnp.logical_and(vi >= lo_row, vi < hi_row)
        plsc.store_compressed(hpi.at[pl.ds(ci, L)], ks, mask=mi)
        plsc.store_compressed(hri.at[pl.ds(ci, L)], vi, mask=mi)
        ci = ci + plsc.all_reduce_population_count(mi)[0]
        return cu, ci

    lax.fori_loop(0, B // L, bucket, (jnp.int32(0), jnp.int32(0)))

    def stage_chunk(startp, wsize):
        cs = []
        for tbl_hbm, buf in ((utT_hbm, ubuf), (itT_hbm, ibuf)):
            for c in range(4):
                cs.append(pltpu.async_copy(
                    tbl_hbm.at[c, slice(None), pl.ds(startp, wsize)],
                    buf.at[c, slice(None), pl.ds(0, wsize)], sem))
        for cpy in cs:
            cpy.wait()

    def process(buf, hp, hr, stage, c0, c1, startp):
        sent = jnp.full((L,), -1, jnp.int32)
        for j in range(CAP // L):
            cpos[pl.ds(j * L, L)] = sent

        def rescan(hv, cc):
            rows = hr[pl.ds(hv * L, L)]
            m = jnp.logical_and(rows >= c0, rows < c1)
            plsc.store_compressed(cpos.at[pl.ds(cc, L)],
                                  hp[pl.ds(hv * L, L)], mask=m)
            plsc.store_compressed(crow.at[pl.ds(cc, L)],
                                  rows - startp, mask=m)
            return cc + plsc.all_reduce_population_count(m)[0]

        lax.fori_loop(0, HCAP // L, rescan, jnp.int32(0))

        for grp in range(CAP // L):
            rows = crow[pl.ds(grp * L, L)]
            slot = grp * L + lax.iota(jnp.int32, L)
            for d in range(D):
                cvec = jnp.full((L,), d >> 3, jnp.int32)
                evec = jnp.full((L,), d & 7, jnp.int32)
                vals = plsc.load_gather(buf, [cvec, evec, rows])
                plsc.store_scatter(stag, [slot, jnp.full((L,), d, jnp.int32)],
                                   vals)
        pltpu.async_copy(
            stag, stage.at[plsc.Indices(cpos, ignored_value=-1)], sem).wait()

    def chunk(ch, carry):
        c0 = lo_row + ch * GB
        c1 = jnp.minimum(c0 + GB, jnp.minimum(hi_row, 999936))
        startp = pl.multiple_of(jnp.minimum(c0, 998912), 128)
        stage_chunk(startp, GB)
        process(ubuf, hpu, hru, ustage, c0, c1, startp)
        process(ibuf, hpi, hri, istage, c0, c1, startp)
        return carry

    lax.fori_loop(0, NCHK, chunk, 0)

    t0 = jnp.minimum(jnp.int32(999936), hi_row)
    ts = pl.multiple_of(jnp.int32(999936), 128)
    stage_chunk(ts, 128)
    process(ubuf, hpu, hru, ustage, t0, hi_row, ts)
    process(ibuf, hpi, hri, istage, t0, hi_row, ts)


@functools.partial(
    pl.kernel,
    out_type=[jax.ShapeDtypeStruct((B, SW), jnp.float32),
              jax.ShapeDtypeStruct((B, SW), jnp.float32)],
    mesh=plsc.VectorSubcoreMesh(core_axis_name="c", subcore_axis_name="s"),
    compiler_params=pltpu.CompilerParams(needs_layout_passes=False),
    scratch_types=[
        pltpu.VMEM((B,), jnp.int32),
        pltpu.VMEM((B,), jnp.int32),
        pltpu.VMEM((HCAP,), jnp.int32),
        pltpu.VMEM((HCAP,), jnp.int32),
        pltpu.VMEM((HCAP,), jnp.int32),
        pltpu.VMEM((HCAP,), jnp.int32),
        pltpu.VMEM((4, 8, GB), jnp.float32),
        pltpu.VMEM((4, 8, GB), jnp.float32),
        pltpu.VMEM((CAP, SW), jnp.float32),
        pltpu.VMEM((CAP,), jnp.int32),
        pltpu.VMEM((CAP,), jnp.int32),
        pltpu.SemaphoreType.DMA,
    ],
)
def _scan_kernel(user_hbm, item_hbm, utT_hbm, itT_hbm, ustage, istage,
                 fu, fi, hpu, hru, hpi, hri, ubuf, ibuf, stag, cpos, crow,
                 sem):
    _scan_body(user_hbm, item_hbm, utT_hbm, itT_hbm, ustage, istage,
               fu, fi, hpu, hru, hpi, hri, ubuf, ibuf, stag, cpos, crow,
               sem)


def _dot_body(u_ref, v_ref, o_ref):
    o_ref[...] = jnp.sum(u_ref[:, :D] * v_ref[:, :D], axis=1)


_dot_kernel = pl.pallas_call(
    _dot_body,
    out_shape=jax.ShapeDtypeStruct((B,), jnp.float32),
    grid=(16,),
    in_specs=[
        pl.BlockSpec((B // 16, SW), lambda i: (i, 0)),
        pl.BlockSpec((B // 16, SW), lambda i: (i, 0)),
    ],
    out_specs=pl.BlockSpec((B // 16,), lambda i: (i,)),
)


def kernel(user, item, user_table, item_table):
    u = user.astype(jnp.int32)
    it = item.astype(jnp.int32)
    utT = user_table.T.reshape(4, 8, NR)
    itT = item_table.T.reshape(4, 8, NR)
    ustage, istage = _scan_kernel(u, it, utT, itT)
    return _dot_kernel(ustage, istage)


# speedup vs baseline: 1.8935x; 1.2840x over previous
import functools

import jax
import jax.numpy as jnp
from jax import lax
from jax.experimental import pallas as pl
from jax.experimental.pallas import tpu as pltpu
from jax.experimental.pallas import tpu_sc as plsc

B = 16384
D = 32
NC = 2
NS = 16
NW = NC * NS
L = 16
NR = 1000000
GB = 1024
NCHK = 31
HCAP = 768
CAP = 64
SW = 128


def _scan_body(user_hbm, item_hbm, utT_hbm, itT_hbm, ustage, istage,
               fu, fi, hpu, hru, hpi, hri, ubuf, ibuf, stag, cpos, crow,
               sem):
    wid = lax.axis_index("s") * NC + lax.axis_index("c")
    lo_g = 244 * wid + jnp.minimum(wid, 5)
    ng = jnp.where(wid < 5, 245, 244)
    lo_row = lo_g * 128
    hi_row = jnp.minimum((lo_g + ng) * 128, NR)

    pltpu.sync_copy(user_hbm, fu)
    pltpu.sync_copy(item_hbm, fi)

    neg1 = jnp.full((L,), -1, jnp.int32)
    zero = jnp.zeros((L,), jnp.int32)
    for j in range(HCAP // L):
        hru[pl.ds(j * L, L)] = neg1
        hri[pl.ds(j * L, L)] = neg1
    for j in range(CAP // L):
        crow[pl.ds(j * L, L)] = zero

    def bucket(kk, cnts):
        cu, ci = cnts
        ks = kk * L + lax.iota(jnp.int32, L)
        vu = fu[pl.ds(kk * L, L)]
        mu = jnp.logical_and(vu >= lo_row, vu < hi_row)
        plsc.store_compressed(hpu.at[pl.ds(cu, L)], ks, mask=mu)
        plsc.store_compressed(hru.at[pl.ds(cu, L)], vu, mask=mu)
        cu = cu + plsc.all_reduce_population_count(mu)[0]
        vi = fi[pl.ds(kk * L, L)]
        mi = jnp.logical_and(vi >= lo_row, vi < hi_row)
        plsc.store_compressed(hpi.at[pl.ds(ci, L)], ks, mask=mi)
        plsc.store_compressed(hri.at[pl.ds(ci, L)], vi, mask=mi)
        ci = ci + plsc.all_reduce_population_count(mi)[0]
        return cu, ci

    ncu, nci = lax.fori_loop(0, B // L, bucket,
                             (jnp.int32(0), jnp.int32(0)))

    def stage_chunk(startp, wsize):
        cs = []
        for tbl_hbm, buf in ((utT_hbm, ubuf), (itT_hbm, ibuf)):
            for c in range(4):
                cs.append(pltpu.async_copy(
                    tbl_hbm.at[c, slice(None), pl.ds(startp, wsize)],
                    buf.at[c, slice(None), pl.ds(0, wsize)], sem))
        for cpy in cs:
            cpy.wait()

    def process(buf, hp, hr, nh, stage, c0, c1, startp):
        sent = jnp.full((L,), -1, jnp.int32)
        for j in range(CAP // L):
            cpos[pl.ds(j * L, L)] = sent

        def rescan(hv, cc):
            rows = hr[pl.ds(hv * L, L)]
            m = jnp.logical_and(rows >= c0, rows < c1)
            plsc.store_compressed(cpos.at[pl.ds(cc, L)],
                                  hp[pl.ds(hv * L, L)], mask=m)
            plsc.store_compressed(crow.at[pl.ds(cc, L)],
                                  rows - startp, mask=m)
            return cc + plsc.all_reduce_population_count(m)[0]

        cc = lax.fori_loop(0, (nh + L - 1) // L, rescan, jnp.int32(0))

        def assemble(grp, carry):
            rows = crow[pl.ds(grp * L, L)]
            slot = grp * L + lax.iota(jnp.int32, L)
            for d in range(D):
                cvec = jnp.full((L,), d >> 3, jnp.int32)
                evec = jnp.full((L,), d & 7, jnp.int32)
                vals = plsc.load_gather(buf, [cvec, evec, rows])
                plsc.store_scatter(stag, [slot, jnp.full((L,), d, jnp.int32)],
                                   vals)
            return carry

        lax.fori_loop(0, (cc + L - 1) // L, assemble, 0)
        pltpu.async_copy(
            stag, stage.at[plsc.Indices(cpos, ignored_value=-1)], sem).wait()

    def chunk(ch, carry):
        c0 = lo_row + ch * GB
        c1 = jnp.minimum(c0 + GB, jnp.minimum(hi_row, 999936))
        startp = pl.multiple_of(jnp.minimum(c0, 998912), 128)
        stage_chunk(startp, GB)
        process(ubuf, hpu, hru, ncu, ustage, c0, c1, startp)
        process(ibuf, hpi, hri, nci, istage, c0, c1, startp)
        return carry

    lax.fori_loop(0, NCHK, chunk, 0)

    t0 = jnp.minimum(jnp.int32(999936), hi_row)
    ts = pl.multiple_of(jnp.int32(999936), 128)
    stage_chunk(ts, 128)
    process(ubuf, hpu, hru, ncu, ustage, t0, hi_row, ts)
    process(ibuf, hpi, hri, nci, istage, t0, hi_row, ts)


@functools.partial(
    pl.kernel,
    out_type=[jax.ShapeDtypeStruct((B, SW), jnp.float32),
              jax.ShapeDtypeStruct((B, SW), jnp.float32)],
    mesh=plsc.VectorSubcoreMesh(core_axis_name="c", subcore_axis_name="s"),
    compiler_params=pltpu.CompilerParams(needs_layout_passes=False),
    scratch_types=[
        pltpu.VMEM((B,), jnp.int32),
        pltpu.VMEM((B,), jnp.int32),
        pltpu.VMEM((HCAP,), jnp.int32),
        pltpu.VMEM((HCAP,), jnp.int32),
        pltpu.VMEM((HCAP,), jnp.int32),
        pltpu.VMEM((HCAP,), jnp.int32),
        pltpu.VMEM((4, 8, GB), jnp.float32),
        pltpu.VMEM((4, 8, GB), jnp.float32),
        pltpu.VMEM((CAP, SW), jnp.float32),
        pltpu.VMEM((CAP,), jnp.int32),
        pltpu.VMEM((CAP,), jnp.int32),
        pltpu.SemaphoreType.DMA,
    ],
)
def _scan_kernel(user_hbm, item_hbm, utT_hbm, itT_hbm, ustage, istage,
                 fu, fi, hpu, hru, hpi, hri, ubuf, ibuf, stag, cpos, crow,
                 sem):
    _scan_body(user_hbm, item_hbm, utT_hbm, itT_hbm, ustage, istage,
               fu, fi, hpu, hru, hpi, hri, ubuf, ibuf, stag, cpos, crow,
               sem)


def _dot_body(u_ref, v_ref, o_ref):
    o_ref[...] = jnp.sum(u_ref[:, :D] * v_ref[:, :D], axis=1)


_dot_kernel = pl.pallas_call(
    _dot_body,
    out_shape=jax.ShapeDtypeStruct((B,), jnp.float32),
    grid=(16,),
    in_specs=[
        pl.BlockSpec((B // 16, SW), lambda i: (i, 0)),
        pl.BlockSpec((B // 16, SW), lambda i: (i, 0)),
    ],
    out_specs=pl.BlockSpec((B // 16,), lambda i: (i,)),
)


def kernel(user, item, user_table, item_table):
    u = user.astype(jnp.int32)
    it = item.astype(jnp.int32)
    utT = user_table.T.reshape(4, 8, NR)
    itT = item_table.T.reshape(4, 8, NR)
    ustage, istage = _scan_kernel(u, it, utT, itT)
    return _dot_kernel(ustage, istage)


# speedup vs baseline: 2.2156x; 1.1701x over previous
import functools

import jax
import jax.numpy as jnp
from jax import lax
from jax.experimental import pallas as pl
from jax.experimental.pallas import tpu as pltpu
from jax.experimental.pallas import tpu_sc as plsc

B = 16384
D = 32
NC = 2
NS = 16
NW = NC * NS
L = 16
NR = 1000000
GB = 512
NCHK = 62
HCAP = 768
CAP = 64
SW = 128


def _scan_body(user_hbm, item_hbm, utT_hbm, itT_hbm, ustage, istage,
               fu, fi, hpu, hru, hpi, hri, ubuf, ibuf, stag, cpos, crow,
               sem0, sem1, ssem):
    wid = lax.axis_index("s") * NC + lax.axis_index("c")
    lo_g = 244 * wid + jnp.minimum(wid, 5)
    ng = jnp.where(wid < 5, 245, 244)
    lo_row = lo_g * 128
    hi_row = jnp.minimum((lo_g + ng) * 128, NR)

    pltpu.sync_copy(user_hbm, fu)
    pltpu.sync_copy(item_hbm, fi)

    neg1 = jnp.full((L,), -1, jnp.int32)
    zero = jnp.zeros((L,), jnp.int32)
    for j in range(HCAP // L):
        hru[pl.ds(j * L, L)] = neg1
        hri[pl.ds(j * L, L)] = neg1
    for j in range(CAP // L):
        crow[pl.ds(j * L, L)] = zero

    def bucket(kk, cnts):
        cu, ci = cnts
        ks = kk * L + lax.iota(jnp.int32, L)
        vu = fu[pl.ds(kk * L, L)]
        mu = jnp.logical_and(vu >= lo_row, vu < hi_row)
        plsc.store_compressed(hpu.at[pl.ds(cu, L)], ks, mask=mu)
        plsc.store_compressed(hru.at[pl.ds(cu, L)], vu, mask=mu)
        cu = cu + plsc.all_reduce_population_count(mu)[0]
        vi = fi[pl.ds(kk * L, L)]
        mi = jnp.logical_and(vi >= lo_row, vi < hi_row)
        plsc.store_compressed(hpi.at[pl.ds(ci, L)], ks, mask=mi)
        plsc.store_compressed(hri.at[pl.ds(ci, L)], vi, mask=mi)
        ci = ci + plsc.all_reduce_population_count(mi)[0]
        return cu, ci

    ncu, nci = lax.fori_loop(0, B // L, bucket,
                             (jnp.int32(0), jnp.int32(0)))

    sems = [sem0, sem1]

    def fire_chunk(bi, startp):
        for tbl_hbm, buf in ((utT_hbm, ubuf), (itT_hbm, ibuf)):
            for c in range(4):
                pltpu.async_copy(
                    tbl_hbm.at[c, slice(None), pl.ds(startp, GB)],
                    buf.at[bi, c], sems[bi])

    def drain_chunk(bi):
        for buf in (ubuf, ibuf):
            for c in range(4):
                pltpu.make_async_copy(
                    utT_hbm.at[c, slice(None), pl.ds(0, GB)],
                    buf.at[bi, c], sems[bi]).wait()

    def stage_chunk(startp, wsize):
        cs = []
        for tbl_hbm, buf in ((utT_hbm, ubuf), (itT_hbm, ibuf)):
            for c in range(4):
                cs.append(pltpu.async_copy(
                    tbl_hbm.at[c, slice(None), pl.ds(startp, wsize)],
                    buf.at[0, c, slice(None), pl.ds(0, wsize)], sems[0]))
        for cpy in cs:
            cpy.wait()

    def process(buf, hp, hr, nh, stage, c0, c1, startp):
        sent = jnp.full((L,), -1, jnp.int32)
        for j in range(CAP // L):
            cpos[pl.ds(j * L, L)] = sent

        def rescan(hv, cc):
            rows = hr[pl.ds(hv * L, L)]
            m = jnp.logical_and(rows >= c0, rows < c1)
            plsc.store_compressed(cpos.at[pl.ds(cc, L)],
                                  hp[pl.ds(hv * L, L)], mask=m)
            plsc.store_compressed(crow.at[pl.ds(cc, L)],
                                  rows - startp, mask=m)
            return cc + plsc.all_reduce_population_count(m)[0]

        cc = lax.fori_loop(0, (nh + L - 1) // L, rescan, jnp.int32(0))

        def assemble(grp, carry):
            rows = crow[pl.ds(grp * L, L)]
            slot = grp * L + lax.iota(jnp.int32, L)
            for d in range(D):
                cvec = jnp.full((L,), d >> 3, jnp.int32)
                evec = jnp.full((L,), d & 7, jnp.int32)
                vals = plsc.load_gather(buf, [cvec, evec, rows])
                plsc.store_scatter(stag, [slot, jnp.full((L,), d, jnp.int32)],
                                   vals)
            return carry

        lax.fori_loop(0, (cc + L - 1) // L, assemble, 0)
        pltpu.async_copy(
            stag, stage.at[plsc.Indices(cpos, ignored_value=-1)], ssem).wait()

    def cbounds(ch):
        c0 = lo_row + ch * GB
        c1 = jnp.minimum(c0 + GB, jnp.minimum(hi_row, 999936))
        startp = pl.multiple_of(
            jnp.minimum(c0, ((NR - GB) // 128) * 128), 128)
        return c0, c1, startp

    def do(ch, bi):
        c0, c1, startp = cbounds(ch)
        drain_chunk(bi)
        process(ubuf.at[bi], hpu, hru, ncu, ustage, c0, c1, startp)
        process(ibuf.at[bi], hpi, hri, nci, istage, c0, c1, startp)

    fire_chunk(0, cbounds(0)[2])

    def chunk(ch, carry):
        even = lax.rem(ch, 2) == 0

        @pl.when(even)
        def _():
            @pl.when(ch + 1 < NCHK)
            def _():
                fire_chunk(1, cbounds(ch + 1)[2])
            do(ch, 0)

        @pl.when(jnp.logical_not(even))
        def _():
            @pl.when(ch + 1 < NCHK)
            def _():
                fire_chunk(0, cbounds(ch + 1)[2])
            do(ch, 1)
        return carry

    lax.fori_loop(0, NCHK, chunk, 0)

    t0 = jnp.minimum(jnp.int32(999936), hi_row)
    ts = pl.multiple_of(jnp.int32(999936), 128)
    stage_chunk(ts, 128)
    process(ubuf.at[0], hpu, hru, ncu, ustage, t0, hi_row, ts)
    process(ibuf.at[0], hpi, hri, nci, istage, t0, hi_row, ts)


@functools.partial(
    pl.kernel,
    out_type=[jax.ShapeDtypeStruct((B, SW), jnp.float32),
              jax.ShapeDtypeStruct((B, SW), jnp.float32)],
    mesh=plsc.VectorSubcoreMesh(core_axis_name="c", subcore_axis_name="s"),
    compiler_params=pltpu.CompilerParams(needs_layout_passes=False),
    scratch_types=[
        pltpu.VMEM((B,), jnp.int32),
        pltpu.VMEM((B,), jnp.int32),
        pltpu.VMEM((HCAP,), jnp.int32),
        pltpu.VMEM((HCAP,), jnp.int32),
        pltpu.VMEM((HCAP,), jnp.int32),
        pltpu.VMEM((HCAP,), jnp.int32),
        pltpu.VMEM((2, 4, 8, GB), jnp.float32),
        pltpu.VMEM((2, 4, 8, GB), jnp.float32),
        pltpu.VMEM((CAP, SW), jnp.float32),
        pltpu.VMEM((CAP,), jnp.int32),
        pltpu.VMEM((CAP,), jnp.int32),
        pltpu.SemaphoreType.DMA,
        pltpu.SemaphoreType.DMA,
        pltpu.SemaphoreType.DMA,
    ],
)
def _scan_kernel(user_hbm, item_hbm, utT_hbm, itT_hbm, ustage, istage,
                 fu, fi, hpu, hru, hpi, hri, ubuf, ibuf, stag, cpos, crow,
                 sem0, sem1, ssem):
    _scan_body(user_hbm, item_hbm, utT_hbm, itT_hbm, ustage, istage,
               fu, fi, hpu, hru, hpi, hri, ubuf, ibuf, stag, cpos, crow,
               sem0, sem1, ssem)


def _dot_body(u_ref, v_ref, o_ref):
    o_ref[...] = jnp.sum(u_ref[:, :D] * v_ref[:, :D], axis=1)


_dot_kernel = pl.pallas_call(
    _dot_body,
    out_shape=jax.ShapeDtypeStruct((B,), jnp.float32),
    grid=(16,),
    in_specs=[
        pl.BlockSpec((B // 16, SW), lambda i: (i, 0)),
        pl.BlockSpec((B // 16, SW), lambda i: (i, 0)),
    ],
    out_specs=pl.BlockSpec((B // 16,), lambda i: (i,)),
)


def kernel(user, item, user_table, item_table):
    u = user.astype(jnp.int32)
    it = item.astype(jnp.int32)
    utT = user_table.T.reshape(4, 8, NR)
    itT = item_table.T.reshape(4, 8, NR)
    ustage, istage = _scan_kernel(u, it, utT, itT)
    return _dot_kernel(ustage, istage)


# speedup vs baseline: 2.4557x; 1.1083x over previous
import functools

import jax
import jax.numpy as jnp
from jax import lax
from jax.experimental import pallas as pl
from jax.experimental.pallas import tpu as pltpu
from jax.experimental.pallas import tpu_sc as plsc

B = 16384
D = 32
NC = 2
NS = 16
NW = NC * NS
L = 16
NR = 1000000
GB = 640
NCHK = 49
HCAP = 768
CAP = 64
SW = 128


def _scan_body(user_hbm, item_hbm, utT_hbm, itT_hbm, ustage, istage,
               fu, fi, hpu, hru, hpi, hri, ubuf, ibuf, stag, cpos, crow,
               sem0, sem1, ssem):
    wid = lax.axis_index("s") * NC + lax.axis_index("c")
    lo_g = 244 * wid + jnp.minimum(wid, 5)
    ng = jnp.where(wid < 5, 245, 244)
    lo_row = lo_g * 128
    hi_row = jnp.minimum((lo_g + ng) * 128, NR)

    pltpu.sync_copy(user_hbm, fu)
    pltpu.sync_copy(item_hbm, fi)

    neg1 = jnp.full((L,), -1, jnp.int32)
    zero = jnp.zeros((L,), jnp.int32)
    for j in range(HCAP // L):
        hru[pl.ds(j * L, L)] = neg1
        hri[pl.ds(j * L, L)] = neg1
    for j in range(CAP // L):
        crow[pl.ds(j * L, L)] = zero

    def bucket(kk, cnts):
        cu, ci = cnts
        ks = kk * L + lax.iota(jnp.int32, L)
        vu = fu[pl.ds(kk * L, L)]
        mu = jnp.logical_and(vu >= lo_row, vu < hi_row)
        plsc.store_compressed(hpu.at[pl.ds(cu, L)], ks, mask=mu)
        plsc.store_compressed(hru.at[pl.ds(cu, L)], vu, mask=mu)
        cu = cu + plsc.all_reduce_population_count(mu)[0]
        vi = fi[pl.ds(kk * L, L)]
        mi = jnp.logical_and(vi >= lo_row, vi < hi_row)
        plsc.store_compressed(hpi.at[pl.ds(ci, L)], ks, mask=mi)
        plsc.store_compressed(hri.at[pl.ds(ci, L)], vi, mask=mi)
        ci = ci + plsc.all_reduce_population_count(mi)[0]
        return cu, ci

    ncu, nci = lax.fori_loop(0, B // L, bucket,
                             (jnp.int32(0), jnp.int32(0)))

    sems = [sem0, sem1]

    def fire_chunk(bi, startp):
        for tbl_hbm, buf in ((utT_hbm, ubuf), (itT_hbm, ibuf)):
            for c in range(4):
                pltpu.async_copy(
                    tbl_hbm.at[c, slice(None), pl.ds(startp, GB)],
                    buf.at[bi, c], sems[bi])

    def drain_chunk(bi):
        for buf in (ubuf, ibuf):
            for c in range(4):
                pltpu.make_async_copy(
                    utT_hbm.at[c, slice(None), pl.ds(0, GB)],
                    buf.at[bi, c], sems[bi]).wait()

    def stage_chunk(startp, wsize):
        cs = []
        for tbl_hbm, buf in ((utT_hbm, ubuf), (itT_hbm, ibuf)):
            for c in range(4):
                cs.append(pltpu.async_copy(
                    tbl_hbm.at[c, slice(None), pl.ds(startp, wsize)],
                    buf.at[0, c, slice(None), pl.ds(0, wsize)], sems[0]))
        for cpy in cs:
            cpy.wait()

    def process(buf, hp, hr, nh, stage, c0, c1, startp):
        sent = jnp.full((L,), -1, jnp.int32)
        for j in range(CAP // L):
            cpos[pl.ds(j * L, L)] = sent

        def rescan(hv, cc):
            rows = hr[pl.ds(hv * L, L)]
            m = jnp.logical_and(rows >= c0, rows < c1)
            plsc.store_compressed(cpos.at[pl.ds(cc, L)],
                                  hp[pl.ds(hv * L, L)], mask=m)
            plsc.store_compressed(crow.at[pl.ds(cc, L)],
                                  rows - startp, mask=m)
            return cc + plsc.all_reduce_population_count(m)[0]

        cc = lax.fori_loop(0, (nh + L - 1) // L, rescan, jnp.int32(0))

        def assemble(grp, carry):
            rows = crow[pl.ds(grp * L, L)]
            slot = grp * L + lax.iota(jnp.int32, L)
            for d in range(D):
                cvec = jnp.full((L,), d >> 3, jnp.int32)
                evec = jnp.full((L,), d & 7, jnp.int32)
                vals = plsc.load_gather(buf, [cvec, evec, rows])
                plsc.store_scatter(stag, [slot, jnp.full((L,), d, jnp.int32)],
                                   vals)
            return carry

        lax.fori_loop(0, (cc + L - 1) // L, assemble, 0)
        pltpu.async_copy(
            stag, stage.at[plsc.Indices(cpos, ignored_value=-1)], ssem).wait()

    def cbounds(ch):
        c0 = lo_row + ch * GB
        c1 = jnp.minimum(c0 + GB, jnp.minimum(hi_row, 999936))
        startp = pl.multiple_of(
            jnp.minimum(c0, ((NR - GB) // 128) * 128), 128)
        return c0, c1, startp

    def do(ch, bi):
        c0, c1, startp = cbounds(ch)
        drain_chunk(bi)
        process(ubuf.at[bi], hpu, hru, ncu, ustage, c0, c1, startp)
        process(ibuf.at[bi], hpi, hri, nci, istage, c0, c1, startp)

    fire_chunk(0, cbounds(0)[2])

    def chunk(ch, carry):
        even = lax.rem(ch, 2) == 0

        @pl.when(even)
        def _():
            @pl.when(ch + 1 < NCHK)
            def _():
                fire_chunk(1, cbounds(ch + 1)[2])
            do(ch, 0)

        @pl.when(jnp.logical_not(even))
        def _():
            @pl.when(ch + 1 < NCHK)
            def _():
                fire_chunk(0, cbounds(ch + 1)[2])
            do(ch, 1)
        return carry

    lax.fori_loop(0, NCHK, chunk, 0)

    t0 = jnp.minimum(jnp.int32(999936), hi_row)
    ts = pl.multiple_of(jnp.int32(999936), 128)
    stage_chunk(ts, 128)
    process(ubuf.at[0], hpu, hru, ncu, ustage, t0, hi_row, ts)
    process(ibuf.at[0], hpi, hri, nci, istage, t0, hi_row, ts)


@functools.partial(
    pl.kernel,
    out_type=[jax.ShapeDtypeStruct((B, SW), jnp.float32),
              jax.ShapeDtypeStruct((B, SW), jnp.float32)],
    mesh=plsc.VectorSubcoreMesh(core_axis_name="c", subcore_axis_name="s"),
    compiler_params=pltpu.CompilerParams(needs_layout_passes=False),
    scratch_types=[
        pltpu.VMEM((B,), jnp.int32),
        pltpu.VMEM((B,), jnp.int32),
        pltpu.VMEM((HCAP,), jnp.int32),
        pltpu.VMEM((HCAP,), jnp.int32),
        pltpu.VMEM((HCAP,), jnp.int32),
        pltpu.VMEM((HCAP,), jnp.int32),
        pltpu.VMEM((2, 4, 8, GB), jnp.float32),
        pltpu.VMEM((2, 4, 8, GB), jnp.float32),
        pltpu.VMEM((CAP, SW), jnp.float32),
        pltpu.VMEM((CAP,), jnp.int32),
        pltpu.VMEM((CAP,), jnp.int32),
        pltpu.SemaphoreType.DMA,
        pltpu.SemaphoreType.DMA,
        pltpu.SemaphoreType.DMA,
    ],
)
def _scan_kernel(user_hbm, item_hbm, utT_hbm, itT_hbm, ustage, istage,
                 fu, fi, hpu, hru, hpi, hri, ubuf, ibuf, stag, cpos, crow,
                 sem0, sem1, ssem):
    _scan_body(user_hbm, item_hbm, utT_hbm, itT_hbm, ustage, istage,
               fu, fi, hpu, hru, hpi, hri, ubuf, ibuf, stag, cpos, crow,
               sem0, sem1, ssem)


def _dot_body(u_ref, v_ref, o_ref):
    o_ref[...] = jnp.sum(u_ref[:, :D] * v_ref[:, :D], axis=1)


_dot_kernel = pl.pallas_call(
    _dot_body,
    out_shape=jax.ShapeDtypeStruct((B,), jnp.float32),
    grid=(16,),
    in_specs=[
        pl.BlockSpec((B // 16, SW), lambda i: (i, 0)),
        pl.BlockSpec((B // 16, SW), lambda i: (i, 0)),
    ],
    out_specs=pl.BlockSpec((B // 16,), lambda i: (i,)),
)


def kernel(user, item, user_table, item_table):
    u = user.astype(jnp.int32)
    it = item.astype(jnp.int32)
    utT = user_table.T.reshape(4, 8, NR)
    itT = item_table.T.reshape(4, 8, NR)
    ustage, istage = _scan_kernel(u, it, utT, itT)
    return _dot_kernel(ustage, istage)
